# baseline (speedup 1.0000x reference)
import functools

import jax
import jax.numpy as jnp
import numpy as np
from jax import lax
from jax.experimental import pallas as pl
from jax.experimental.pallas import tpu as pltpu
from jax.experimental.pallas import tpu_sc as plsc

_T, _B, _D, _G, _S = 32, 64, 256, 32, 8192
_N = _T * _B
_NUMC = 20
_NC, _NS = 2, 16
_NW = _NC * _NS
_TPW = _N // _NW
_EPS = 1e-8

_LOGEPS = float(np.log(np.float32(_EPS)))
_LOGQ = float(np.log(np.float32(1.0 - _EPS * (_S - 1))))

_INTERPRET = False


def _context_body(prev_ref, we_ref, loc0_ref, lc_ref):
    tt = lax.broadcasted_iota(jnp.int32, (_T, _T), 0)
    jj = lax.broadcasted_iota(jnp.int32, (_T, _T), 1)
    wp = (jj >= tt + (_T - _NUMC + 1)).astype(jnp.float32)
    ww = ((jj <= tt) & (jj >= tt - (_NUMC - 1))).astype(jnp.float32)
    acc = jnp.dot(wp, prev_ref[...], preferred_element_type=jnp.float32)
    acc = acc + jnp.dot(ww, we_ref[...], preferred_element_type=jnp.float32)
    lc_ref[...] = loc0_ref[...] + acc / float(_NUMC)


def _context(prev2d, we2d, loc2d):
    cb = 4096
    grid = (_B * _D) // cb
    spec = pl.BlockSpec((_T, cb), lambda i: (0, i))
    return pl.pallas_call(
        _context_body,
        grid=(grid,),
        in_specs=[spec, spec, spec],
        out_specs=spec,
        out_shape=jax.ShapeDtypeStruct((_T, _B * _D), jnp.float32),
        interpret=_INTERPRET,
    )(prev2d, we2d, loc2d)


def _rownorm_body(x_ref, n_ref):
    x = x_ref[...]
    n_ref[...] = jnp.sqrt(jnp.sum(x * x, axis=1, keepdims=True))


def _rownorm(x, rb):
    rows = x.shape[0]
    return pl.pallas_call(
        _rownorm_body,
        grid=(rows // rb,),
        in_specs=[pl.BlockSpec((rb, _D), lambda i: (i, 0))],
        out_specs=pl.BlockSpec((rb, 1), lambda i: (i, 0)),
        out_shape=jax.ShapeDtypeStruct((rows, 1), jnp.float32),
        interpret=_INTERPRET,
    )(x)


def _worker_id():
    return lax.axis_index("s") * _NC + lax.axis_index("c")


def _sc_select_body(sc_hbm, idx_hbm, lc_hbm, n1_hbm, n2_hbm, out_hbm,
                    idx_v, lc_v, n1_v, n2_v, rows_v, out_v, sem0, sem1):
    wid = _worker_id()
    base = wid * _TPW
    pltpu.sync_copy(idx_hbm.at[pl.ds(base * _G, _TPW * _G)],
                    idx_v.at[pl.ds(0, _TPW * _G)])
    pltpu.sync_copy(lc_hbm.at[pl.ds(base * _D, _TPW * _D)], lc_v)
    pltpu.sync_copy(n1_hbm.at[pl.ds(base, _TPW)], n1_v)
    pltpu.sync_copy(n2_hbm, n2_v)
    iota = lax.iota(jnp.int32, 16)
    sems = (sem0, sem1)

    def load_idx(t):
        i0 = jnp.clip(idx_v[pl.ds(t * _G, 16)], 0, _S - 1)
        i1 = jnp.clip(idx_v[pl.ds(t * _G + 16, 16)], 0, _S - 1)
        return i0, i1

    def issue(t, buf):
        i0, i1 = load_idx(t)
        pltpu.async_copy(sc_hbm.at[i0], rows_v.at[buf, pl.ds(0, 16)],
                         sems[buf])
        pltpu.async_copy(sc_hbm.at[i1], rows_v.at[buf, pl.ds(16, 16)],
                         sems[buf])

    def compute(t, buf):
        i0, i1 = load_idx(t)
        pltpu.make_async_copy(sc_hbm.at[i0], rows_v.at[buf, pl.ds(0, 16)],
                              sems[buf]).wait()
        pltpu.make_async_copy(sc_hbm.at[i1], rows_v.at[buf, pl.ds(16, 16)],
                              sems[buf]).wait()
        tvec = jnp.broadcast_to(t, (16,))
        n1b = plsc.load_gather(n1_v, [tvec])
        lcs = [lc_v[pl.ds(t * _D + k * 16, 16)] for k in range(16)]
        cs = []
        for half, iv in enumerate((i0, i1)):
            n2h = plsc.load_gather(n2_v, [iv])

            def g_body(g, dv, _half=half):
                acc = rows_v[buf, _half * 16 + g, pl.ds(0, 16)] * lcs[0]
                for k in range(1, 16):
                    acc = acc + (rows_v[buf, _half * 16 + g,
                                        pl.ds(k * 16, 16)] * lcs[k])
                return jnp.where(iota == g, jnp.sum(acc), dv)

            dv = lax.fori_loop(0, 16, g_body, jnp.zeros((16,), jnp.float32),
                               unroll=4)
            cs.append(dv / jnp.maximum(n1b * n2h, _EPS))
        m = jnp.maximum(jnp.max(cs[0]), jnp.max(cs[1]))
        a0 = jnp.min(jnp.where(cs[0] == m, iota, 64))
        a1 = jnp.min(jnp.where(cs[1] == m, iota + 16, 64))
        arg = jnp.minimum(a0, a1)
        s0 = jnp.max(jnp.where(iota == arg, i0, -1))
        s1 = jnp.max(jnp.where(iota + 16 == arg, i1, -1))
        fs = jnp.maximum(s0, s1)
        plsc.store_scatter(out_v, [tvec], jnp.broadcast_to(fs, (16,)),
                           mask=iota == 0)

    issue(0, 0)

    def pair(i, carry):
        t0 = 2 * i
        issue(t0 + 1, 1)
        compute(t0, 0)
        issue(t0 + 2, 0)
        compute(t0 + 1, 1)
        return carry

    lax.fori_loop(0, _TPW // 2, pair, 0)
    i0, i1 = load_idx(_TPW)
    pltpu.make_async_copy(sc_hbm.at[i0], rows_v.at[0, pl.ds(0, 16)],
                          sem0).wait()
    pltpu.make_async_copy(sc_hbm.at[i1], rows_v.at[0, pl.ds(16, 16)],
                          sem0).wait()
    pltpu.sync_copy(out_v, out_hbm.at[pl.ds(base, _TPW)])


def _sc_select(sc_table, idx_flat, lc_flat1d, n1_flat, n2_flat):
    mesh = plsc.VectorSubcoreMesh(core_axis_name="c", subcore_axis_name="s",
                                  num_cores=_NC, num_subcores=_NS)
    fn = pl.kernel(
        _sc_select_body,
        out_type=jax.ShapeDtypeStruct((_N,), jnp.int32),
        mesh=mesh,
        scratch_types=[
            pltpu.VMEM(((_TPW + 1) * _G,), jnp.int32),
            pltpu.VMEM((_TPW * _D,), jnp.float32),
            pltpu.VMEM((_TPW,), jnp.float32),
            pltpu.VMEM((_S,), jnp.float32),
            pltpu.VMEM((2, _G, _D), jnp.float32),
            pltpu.VMEM((_TPW,), jnp.int32),
            pltpu.SemaphoreType.DMA,
            pltpu.SemaphoreType.DMA,
        ],
        compiler_params=pltpu.CompilerParams(needs_layout_passes=False),
        interpret=_INTERPRET,
    )
    return fn(sc_table, idx_flat, lc_flat1d, n1_flat, n2_flat)


def _fill_body(out_ref):
    out_ref[...] = jnp.full(out_ref.shape, _LOGEPS, jnp.float32)


def _fill():
    pr = 64
    return pl.pallas_call(
        _fill_body,
        grid=(_N // pr,),
        in_specs=[],
        out_specs=pl.BlockSpec((pr, _S), lambda i: (i, 0)),
        out_shape=jax.ShapeDtypeStruct((_N, _S), jnp.float32),
        interpret=_INTERPRET,
    )()


def _sc_scatter_body(fs_hbm, preds_hbm, fs_v, flat_v, val_v, sem):
    wid = _worker_id()
    base = wid * _TPW
    pltpu.sync_copy(fs_hbm.at[pl.ds(base, _TPW)], fs_v)
    iota = lax.iota(jnp.int32, 16)
    for c in range(_TPW // 16):
        iv = fs_v[pl.ds(c * 16, 16)]
        tok = iota + (base + c * 16)
        flat_v[pl.ds(c * 16, 16)] = tok * _S + iv
        val_v[pl.ds(c * 16, 16)] = jnp.full((16,), _LOGQ, jnp.float32)
    pltpu.async_copy(val_v, preds_hbm.at[flat_v], sem).wait()


def _sc_scatter(fs, preds_ref):
    mesh = plsc.VectorSubcoreMesh(core_axis_name="c", subcore_axis_name="s",
                                  num_cores=_NC, num_subcores=_NS)
    fn = pl.kernel(
        _sc_scatter_body,
        out_type=(),
        mesh=mesh,
        scratch_types=[
            pltpu.VMEM((_TPW,), jnp.int32),
            pltpu.VMEM((_TPW,), jnp.int32),
            pltpu.VMEM((_TPW,), jnp.float32),
            pltpu.SemaphoreType.DMA,
        ],
        compiler_params=pltpu.CompilerParams(needs_layout_passes=False),
        interpret=_INTERPRET,
    )
    fn(fs, preds_ref)


def kernel(word_embeddings, prev_word_embeddings, location_context, SC,
           all_sense_neighbours):
    prev2d = prev_word_embeddings.reshape(_T, _B * _D)
    we2d = word_embeddings.reshape(_T, _B * _D)
    loc2d = location_context.reshape(_T, _B * _D)

    lc2d = _context(prev2d, we2d, loc2d)
    lc_flat = lc2d.reshape(_N, _D)

    n1 = _rownorm(lc_flat, 256).reshape(_N)
    n2 = _rownorm(SC, 1024).reshape(_S)

    idx_flat = all_sense_neighbours.reshape(_N * _G)
    fs = _sc_select(SC, idx_flat, lc_flat.reshape(_N * _D), n1, n2)

    filled = _fill()
    preds_ref = jax.new_ref(filled.reshape(_N * _S))
    _sc_scatter(fs, preds_ref)
    preds = preds_ref[...].reshape(_N, _S)
    return (preds, lc2d.reshape(_T, _B, _D))


# speedup vs baseline: 1.2817x; 1.2817x over previous
import functools

import jax
import jax.numpy as jnp
import numpy as np
from jax import lax
from jax.experimental import pallas as pl
from jax.experimental.pallas import tpu as pltpu
from jax.experimental.pallas import tpu_sc as plsc

_T, _B, _D, _G, _S = 32, 64, 256, 32, 8192
_N = _T * _B
_NUMC = 20
_NC, _NS = 2, 16
_NW = _NC * _NS
_TPW = _N // _NW
_EPS = 1e-8

_LOGEPS = float(np.log(np.float32(_EPS)))
_LOGQ = float(np.log(np.float32(1.0 - _EPS * (_S - 1))))

_INTERPRET = False


def _context_body(prev_ref, we_ref, loc0_ref, lc_ref):
    tt = lax.broadcasted_iota(jnp.int32, (_T, _T), 0)
    jj = lax.broadcasted_iota(jnp.int32, (_T, _T), 1)
    wp = (jj >= tt + (_T - _NUMC + 1)).astype(jnp.float32)
    ww = ((jj <= tt) & (jj >= tt - (_NUMC - 1))).astype(jnp.float32)
    acc = jnp.dot(wp, prev_ref[...], preferred_element_type=jnp.float32)
    acc = acc + jnp.dot(ww, we_ref[...], preferred_element_type=jnp.float32)
    lc_ref[...] = loc0_ref[...] + acc / float(_NUMC)


def _context(prev2d, we2d, loc2d):
    cb = 4096
    grid = (_B * _D) // cb
    spec = pl.BlockSpec((_T, cb), lambda i: (0, i))
    return pl.pallas_call(
        _context_body,
        grid=(grid,),
        in_specs=[spec, spec, spec],
        out_specs=spec,
        out_shape=jax.ShapeDtypeStruct((_T, _B * _D), jnp.float32),
        interpret=_INTERPRET,
    )(prev2d, we2d, loc2d)


def _rownorm_body(x_ref, n_ref):
    x = x_ref[...]
    n_ref[...] = jnp.sqrt(jnp.sum(x * x, axis=1, keepdims=True))


def _rownorm(x, rb):
    rows = x.shape[0]
    return pl.pallas_call(
        _rownorm_body,
        grid=(rows // rb,),
        in_specs=[pl.BlockSpec((rb, _D), lambda i: (i, 0))],
        out_specs=pl.BlockSpec((rb, 1), lambda i: (i, 0)),
        out_shape=jax.ShapeDtypeStruct((rows, 1), jnp.float32),
        interpret=_INTERPRET,
    )(x)


def _worker_id():
    return lax.axis_index("s") * _NC + lax.axis_index("c")


def _sc_select_body(sc_hbm, idx_hbm, lc_hbm, n1_hbm, n2_hbm, preds_hbm,
                    idx_v, lc_v, n1_v, n2_v, rows_v, row2_v,
                    sem0, sem1, osem0, osem1):
    wid = _worker_id()
    base = wid * _TPW
    pltpu.sync_copy(idx_hbm.at[pl.ds(base * _G, _TPW * _G)],
                    idx_v.at[pl.ds(0, _TPW * _G)])
    pltpu.sync_copy(lc_hbm.at[pl.ds(base * _D, _TPW * _D)], lc_v)
    pltpu.sync_copy(n1_hbm.at[pl.ds(base, _TPW)], n1_v)
    pltpu.sync_copy(n2_hbm, n2_v)
    iota = lax.iota(jnp.int32, 16)
    sems = (sem0, sem1)
    osems = (osem0, osem1)
    lane0 = iota == 0
    logq_vec = jnp.full((16,), _LOGQ, jnp.float32)
    logeps_vec = jnp.full((16,), _LOGEPS, jnp.float32)

    def fill_body(j, carry):
        row2_v[pl.ds(j * 16, 16)] = logeps_vec
        row2_v[pl.ds(_S + j * 16, 16)] = logeps_vec
        return carry

    lax.fori_loop(0, _S // 16, fill_body, 0, unroll=8)

    def load_idx(t):
        i0 = jnp.clip(idx_v[pl.ds(t * _G, 16)], 0, _S - 1)
        i1 = jnp.clip(idx_v[pl.ds(t * _G + 16, 16)], 0, _S - 1)
        return i0, i1

    def issue(t, buf):
        i0, i1 = load_idx(t)
        pltpu.async_copy(sc_hbm.at[i0], rows_v.at[buf, pl.ds(0, 16)],
                         sems[buf])
        pltpu.async_copy(sc_hbm.at[i1], rows_v.at[buf, pl.ds(16, 16)],
                         sems[buf])

    def compute(t, buf, prev_fs, first):
        i0, i1 = load_idx(t)
        pltpu.make_async_copy(sc_hbm.at[i0], rows_v.at[buf, pl.ds(0, 16)],
                              sems[buf]).wait()
        pltpu.make_async_copy(sc_hbm.at[i1], rows_v.at[buf, pl.ds(16, 16)],
                              sems[buf]).wait()
        tvec = jnp.broadcast_to(t, (16,))
        n1b = plsc.load_gather(n1_v, [tvec])
        lcs = [lc_v[pl.ds(t * _D + k * 16, 16)] for k in range(16)]
        cs = []
        for half, iv in enumerate((i0, i1)):
            n2h = plsc.load_gather(n2_v, [iv])

            def g_body(g, dv, _half=half):
                acc = rows_v[buf, _half * 16 + g, pl.ds(0, 16)] * lcs[0]
                for k in range(1, 16):
                    acc = acc + (rows_v[buf, _half * 16 + g,
                                        pl.ds(k * 16, 16)] * lcs[k])
                return jnp.where(iota == g, jnp.sum(acc), dv)

            dv = lax.fori_loop(0, 16, g_body, jnp.zeros((16,), jnp.float32),
                               unroll=4)
            cs.append(dv / jnp.maximum(n1b * n2h, _EPS))
        m = jnp.maximum(jnp.max(cs[0]), jnp.max(cs[1]))
        a0 = jnp.min(jnp.where(cs[0] == m, iota, 64))
        a1 = jnp.min(jnp.where(cs[1] == m, iota + 16, 64))
        arg = jnp.minimum(a0, a1)
        s0 = jnp.max(jnp.where(iota == arg, i0, -1))
        s1 = jnp.max(jnp.where(iota + 16 == arg, i1, -1))
        fs = jnp.maximum(s0, s1)
        if not first:
            pltpu.make_async_copy(row2_v.at[pl.ds(buf * _S, _S)],
                                  preds_hbm.at[pl.ds((base + t - 2) * _S, _S)],
                                  osems[buf]).wait()
            plsc.store_scatter(row2_v,
                               [jnp.broadcast_to(buf * _S + prev_fs, (16,))],
                               logeps_vec, mask=lane0)
        plsc.store_scatter(row2_v, [jnp.broadcast_to(buf * _S + fs, (16,))],
                           logq_vec, mask=lane0)
        pltpu.async_copy(row2_v.at[pl.ds(buf * _S, _S)],
                         preds_hbm.at[pl.ds((base + t) * _S, _S)],
                         osems[buf])
        return fs

    issue(0, 0)
    issue(1, 1)
    fs0 = compute(0, 0, 0, True)
    issue(2, 0)
    fs1 = compute(1, 1, 0, True)

    def pair(i, carry):
        pfs0, pfs1 = carry
        t0 = 2 * i
        issue(t0 + 1, 1)
        nfs0 = compute(t0, 0, pfs0, False)
        issue(t0 + 2, 0)
        nfs1 = compute(t0 + 1, 1, pfs1, False)
        return (nfs0, nfs1)

    lax.fori_loop(1, _TPW // 2, pair, (fs0, fs1))
    i0, i1 = load_idx(_TPW)
    pltpu.make_async_copy(sc_hbm.at[i0], rows_v.at[0, pl.ds(0, 16)],
                          sem0).wait()
    pltpu.make_async_copy(sc_hbm.at[i1], rows_v.at[0, pl.ds(16, 16)],
                          sem0).wait()
    pltpu.make_async_copy(row2_v.at[pl.ds(0, _S)],
                          preds_hbm.at[pl.ds((base + _TPW - 2) * _S, _S)],
                          osem0).wait()
    pltpu.make_async_copy(row2_v.at[pl.ds(_S, _S)],
                          preds_hbm.at[pl.ds((base + _TPW - 1) * _S, _S)],
                          osem1).wait()


def _sc_select(sc_table, idx_flat, lc_flat1d, n1_flat, n2_flat):
    mesh = plsc.VectorSubcoreMesh(core_axis_name="c", subcore_axis_name="s",
                                  num_cores=_NC, num_subcores=_NS)
    fn = pl.kernel(
        _sc_select_body,
        out_type=jax.ShapeDtypeStruct((_N * _S,), jnp.float32),
        mesh=mesh,
        scratch_types=[
            pltpu.VMEM(((_TPW + 1) * _G,), jnp.int32),
            pltpu.VMEM((_TPW * _D,), jnp.float32),
            pltpu.VMEM((_TPW,), jnp.float32),
            pltpu.VMEM((_S,), jnp.float32),
            pltpu.VMEM((2, _G, _D), jnp.float32),
            pltpu.VMEM((2 * _S,), jnp.float32),
            pltpu.SemaphoreType.DMA,
            pltpu.SemaphoreType.DMA,
            pltpu.SemaphoreType.DMA,
            pltpu.SemaphoreType.DMA,
        ],
        compiler_params=pltpu.CompilerParams(needs_layout_passes=False),
        interpret=_INTERPRET,
    )
    return fn(sc_table, idx_flat, lc_flat1d, n1_flat, n2_flat)


def kernel(word_embeddings, prev_word_embeddings, location_context, SC,
           all_sense_neighbours):
    prev2d = prev_word_embeddings.reshape(_T, _B * _D)
    we2d = word_embeddings.reshape(_T, _B * _D)
    loc2d = location_context.reshape(_T, _B * _D)

    lc2d = _context(prev2d, we2d, loc2d)
    lc_flat = lc2d.reshape(_N, _D)

    n1 = _rownorm(lc_flat, 256).reshape(_N)
    n2 = _rownorm(SC, 1024).reshape(_S)

    idx_flat = all_sense_neighbours.reshape(_N * _G)
    preds = _sc_select(SC, idx_flat, lc_flat.reshape(_N * _D), n1, n2)
    return (preds.reshape(_N, _S), lc2d.reshape(_T, _B, _D))


# speedup vs baseline: 1.8145x; 1.4157x over previous
import functools

import jax
import jax.numpy as jnp
import numpy as np
from jax import lax
from jax.experimental import pallas as pl
from jax.experimental.pallas import tpu as pltpu
from jax.experimental.pallas import tpu_sc as plsc

_T, _B, _D, _G, _S = 32, 64, 256, 32, 8192
_N = _T * _B
_NUMC = 20
_NC, _NS = 2, 16
_NW = _NC * _NS
_TPW = _N // _NW
_EPS = 1e-8

_LOGEPS = float(np.log(np.float32(_EPS)))
_LOGQ = float(np.log(np.float32(1.0 - _EPS * (_S - 1))))

_INTERPRET = False


def _context_body(prev_ref, we_ref, loc0_ref, lc_ref):
    tt = lax.broadcasted_iota(jnp.int32, (_T, _T), 0)
    jj = lax.broadcasted_iota(jnp.int32, (_T, _T), 1)
    wp = (jj >= tt + (_T - _NUMC + 1)).astype(jnp.float32)
    ww = ((jj <= tt) & (jj >= tt - (_NUMC - 1))).astype(jnp.float32)
    acc = jnp.dot(wp, prev_ref[...], preferred_element_type=jnp.float32)
    acc = acc + jnp.dot(ww, we_ref[...], preferred_element_type=jnp.float32)
    lc_ref[...] = loc0_ref[...] + acc / float(_NUMC)


def _context(prev2d, we2d, loc2d):
    cb = 4096
    grid = (_B * _D) // cb
    spec = pl.BlockSpec((_T, cb), lambda i: (0, i))
    return pl.pallas_call(
        _context_body,
        grid=(grid,),
        in_specs=[spec, spec, spec],
        out_specs=spec,
        out_shape=jax.ShapeDtypeStruct((_T, _B * _D), jnp.float32),
        interpret=_INTERPRET,
    )(prev2d, we2d, loc2d)


def _rownorm_body(x_ref, n_ref):
    x = x_ref[...]
    n_ref[...] = jnp.sqrt(jnp.sum(x * x, axis=1, keepdims=True))


def _rownorm(x, rb):
    rows = x.shape[0]
    return pl.pallas_call(
        _rownorm_body,
        grid=(rows // rb,),
        in_specs=[pl.BlockSpec((rb, _D), lambda i: (i, 0))],
        out_specs=pl.BlockSpec((rb, 1), lambda i: (i, 0)),
        out_shape=jax.ShapeDtypeStruct((rows, 1), jnp.float32),
        interpret=_INTERPRET,
    )(x)


def _worker_id():
    return lax.axis_index("s") * _NC + lax.axis_index("c")


def _sc_select_body(sc_hbm, idx_hbm, lc_hbm, n1_hbm, n2_hbm, preds_hbm,
                    idx_v, lc_v, n1_v, n2_v, rows_v, row2_v,
                    sem0, sem1, osem0, osem1):
    wid = _worker_id()
    base = wid * _TPW
    pltpu.sync_copy(idx_hbm.at[pl.ds(base * _G, _TPW * _G)],
                    idx_v.at[pl.ds(0, _TPW * _G)])
    pltpu.sync_copy(lc_hbm.at[pl.ds(base * _D, _TPW * _D)], lc_v)
    pltpu.sync_copy(n1_hbm.at[pl.ds(base, _TPW)], n1_v)
    pltpu.sync_copy(n2_hbm, n2_v)
    iota = lax.iota(jnp.int32, 16)
    sems = (sem0, sem1)
    osems = (osem0, osem1)
    lane0 = iota == 0
    logq_vec = jnp.full((16,), _LOGQ, jnp.float32)
    logeps_vec = jnp.full((16,), _LOGEPS, jnp.float32)

    def fill_body(j, carry):
        row2_v[0, pl.ds(j * 16, 16)] = logeps_vec
        row2_v[1, pl.ds(j * 16, 16)] = logeps_vec
        return carry

    lax.fori_loop(0, _S // 16, fill_body, 0, unroll=8)

    def load_idx(t):
        i0 = jnp.clip(idx_v[pl.ds(t * _G, 16)], 0, _S - 1)
        i1 = jnp.clip(idx_v[pl.ds(t * _G + 16, 16)], 0, _S - 1)
        return i0, i1

    def issue(t, buf):
        i0, i1 = load_idx(t)
        pltpu.async_copy(sc_hbm.at[i0], rows_v.at[buf, pl.ds(0, 16)],
                         sems[buf])
        pltpu.async_copy(sc_hbm.at[i1], rows_v.at[buf, pl.ds(16, 16)],
                         sems[buf])

    def compute(t, buf, prev_fs, first):
        i0, i1 = load_idx(t)
        pltpu.make_async_copy(sc_hbm.at[i0], rows_v.at[buf, pl.ds(0, 16)],
                              sems[buf]).wait()
        pltpu.make_async_copy(sc_hbm.at[i1], rows_v.at[buf, pl.ds(16, 16)],
                              sems[buf]).wait()
        tvec = jnp.broadcast_to(t, (16,))
        n1b = plsc.load_gather(n1_v, [tvec])
        lcs = [lc_v[pl.ds(t * _D + k * 16, 16)] for k in range(16)]
        cs = []
        for half, iv in enumerate((i0, i1)):
            n2h = plsc.load_gather(n2_v, [iv])

            def g_body(g, dv, _half=half):
                acc = rows_v[buf, _half * 16 + g, pl.ds(0, 16)] * lcs[0]
                for k in range(1, 16):
                    acc = acc + (rows_v[buf, _half * 16 + g,
                                        pl.ds(k * 16, 16)] * lcs[k])
                return jnp.where(iota == g, jnp.sum(acc), dv)

            dv = lax.fori_loop(0, 16, g_body, jnp.zeros((16,), jnp.float32),
                               unroll=4)
            cs.append(dv / jnp.maximum(n1b * n2h, _EPS))
        m = jnp.maximum(jnp.max(cs[0]), jnp.max(cs[1]))
        a0 = jnp.min(jnp.where(cs[0] == m, iota, 64))
        a1 = jnp.min(jnp.where(cs[1] == m, iota + 16, 64))
        arg = jnp.minimum(a0, a1)
        s0 = jnp.max(jnp.where(iota == arg, i0, -1))
        s1 = jnp.max(jnp.where(iota + 16 == arg, i1, -1))
        fs = jnp.maximum(s0, s1)
        if not first:
            pltpu.make_async_copy(row2_v.at[pl.ds(buf, 1), :],
                                  preds_hbm.at[pl.ds(base + t - 2, 1), :],
                                  osems[buf]).wait()
            plsc.store_scatter(row2_v,
                               [jnp.broadcast_to(buf, (16,)),
                                jnp.broadcast_to(prev_fs, (16,))],
                               logeps_vec, mask=lane0)
        plsc.store_scatter(row2_v, [jnp.broadcast_to(buf, (16,)),
                                    jnp.broadcast_to(fs, (16,))],
                           logq_vec, mask=lane0)
        pltpu.async_copy(row2_v.at[pl.ds(buf, 1), :],
                         preds_hbm.at[pl.ds(base + t, 1), :],
                         osems[buf])
        return fs

    issue(0, 0)
    issue(1, 1)
    fs0 = compute(0, 0, 0, True)
    issue(2, 0)
    fs1 = compute(1, 1, 0, True)

    def pair(i, carry):
        pfs0, pfs1 = carry
        t0 = 2 * i
        issue(t0 + 1, 1)
        nfs0 = compute(t0, 0, pfs0, False)
        issue(t0 + 2, 0)
        nfs1 = compute(t0 + 1, 1, pfs1, False)
        return (nfs0, nfs1)

    lax.fori_loop(1, _TPW // 2, pair, (fs0, fs1))
    i0, i1 = load_idx(_TPW)
    pltpu.make_async_copy(sc_hbm.at[i0], rows_v.at[0, pl.ds(0, 16)],
                          sem0).wait()
    pltpu.make_async_copy(sc_hbm.at[i1], rows_v.at[0, pl.ds(16, 16)],
                          sem0).wait()
    pltpu.make_async_copy(row2_v.at[pl.ds(0, 1), :],
                          preds_hbm.at[pl.ds(base + _TPW - 2, 1), :],
                          osem0).wait()
    pltpu.make_async_copy(row2_v.at[pl.ds(1, 1), :],
                          preds_hbm.at[pl.ds(base + _TPW - 1, 1), :],
                          osem1).wait()


def _sc_select(sc_table, idx_flat, lc_flat1d, n1_flat, n2_flat):
    mesh = plsc.VectorSubcoreMesh(core_axis_name="c", subcore_axis_name="s",
                                  num_cores=_NC, num_subcores=_NS)
    fn = pl.kernel(
        _sc_select_body,
        out_type=jax.ShapeDtypeStruct((_N, _S), jnp.float32),
        mesh=mesh,
        scratch_types=[
            pltpu.VMEM(((_TPW + 1) * _G,), jnp.int32),
            pltpu.VMEM((_TPW * _D,), jnp.float32),
            pltpu.VMEM((_TPW,), jnp.float32),
            pltpu.VMEM((_S,), jnp.float32),
            pltpu.VMEM((2, _G, _D), jnp.float32),
            pltpu.VMEM((2, _S), jnp.float32),
            pltpu.SemaphoreType.DMA,
            pltpu.SemaphoreType.DMA,
            pltpu.SemaphoreType.DMA,
            pltpu.SemaphoreType.DMA,
        ],
        compiler_params=pltpu.CompilerParams(needs_layout_passes=False),
        interpret=_INTERPRET,
    )
    return fn(sc_table, idx_flat, lc_flat1d, n1_flat, n2_flat)


def kernel(word_embeddings, prev_word_embeddings, location_context, SC,
           all_sense_neighbours):
    prev2d = prev_word_embeddings.reshape(_T, _B * _D)
    we2d = word_embeddings.reshape(_T, _B * _D)
    loc2d = location_context.reshape(_T, _B * _D)

    lc2d = _context(prev2d, we2d, loc2d)
    lc_flat = lc2d.reshape(_N, _D)

    n1 = _rownorm(lc_flat, 256).reshape(_N)
    n2 = _rownorm(SC, 1024).reshape(_S)

    idx_flat = all_sense_neighbours.reshape(_N * _G)
    preds = _sc_select(SC, idx_flat, lc_flat.reshape(_N * _D), n1, n2)
    return (preds, lc2d.reshape(_T, _B, _D))
